# Initial kernel scaffold; baseline (speedup 1.0000x reference)
import jax
import jax.numpy as jnp
from jax.experimental import pallas as pl


def kernel(x, edge_index, W, attn):
    raise NotImplementedError("write your pallas kernel here")


# speedup vs baseline: 55.1968x; 55.1968x over previous
import functools

import jax
import jax.numpy as jnp
from jax import lax
from jax.experimental import pallas as pl
from jax.experimental.pallas import tpu as pltpu
from jax.experimental.pallas import tpu_sc as plsc

IN_DIM = 128
OUT_DIM = 128
N_HEADS = 4
HEAD_DIM = OUT_DIM // N_HEADS
B = 2
N = 10000
E = 320000

NC = 2
NS = 16
NW = NC * NS
L = 16
EW = E // NW
NPAD = 10240
NB = NPAD // NW

CH = 80
NCH = EW // CH
SCH = 2000
NSCH = EW // SCH
CPS = SCH // CH

CH1 = 400
NCH1 = EW // CH1

RPT = NPAD // NS
ZR = 64

AF = 2 * N_HEADS

_mesh = plsc.VectorSubcoreMesh(
    core_axis_name="c", subcore_axis_name="s", num_cores=NC, num_subcores=NS)


def _tc_proj_body(x_ref, wt_ref, al_ref, h_ref, a_ref):
    xb = x_ref[0]
    hb = jnp.dot(xb, wt_ref[...], preferred_element_type=jnp.float32)
    h_ref[0] = hb
    a_ref[0] = jnp.dot(hb, al_ref[...], preferred_element_type=jnp.float32)


def _tc_proj(x, wt, al):
    blk = 400
    grid = (B, N // blk)
    return pl.pallas_call(
        _tc_proj_body,
        grid=grid,
        in_specs=[
            pl.BlockSpec((1, blk, IN_DIM), lambda b, i: (b, i, 0)),
            pl.BlockSpec((IN_DIM, OUT_DIM), lambda b, i: (0, 0)),
            pl.BlockSpec((OUT_DIM, AF), lambda b, i: (0, 0)),
        ],
        out_specs=[
            pl.BlockSpec((1, blk, OUT_DIM), lambda b, i: (b, i, 0)),
            pl.BlockSpec((1, blk, AF), lambda b, i: (b, i, 0)),
        ],
        out_shape=[
            jax.ShapeDtypeStruct((B, N, OUT_DIM), jnp.float32),
            jax.ShapeDtypeStruct((B, N, AF), jnp.float32),
        ],
    )(x, wt, al)


def _tc_add_body(a_ref, b_ref, o_ref):
    o_ref[...] = a_ref[...] + b_ref[...]


def _tc_add(pa, pb):
    blk = 400
    grid = (B, N // blk)
    in_spec = pl.BlockSpec((1, blk, OUT_DIM), lambda b, i: (b, i, 0))
    return pl.pallas_call(
        _tc_add_body,
        grid=grid,
        in_specs=[in_spec, in_spec],
        out_specs=pl.BlockSpec((1, blk, OUT_DIM), lambda b, i: (b, i, 0)),
        out_shape=jax.ShapeDtypeStruct((B, N, OUT_DIM), jnp.float32),
    )(pa, pb)


def _sc_scores_body(alphas_hbm, src_hbm, dst_hbm,
                    p0_hbm, mslab_hbm, uslab_hbm,
                    alpha_v, srcf_v, dstf_v, m_v, u_v, p0b_v, sem):
    del sem
    cid = lax.axis_index("c")
    sid = lax.axis_index("s")
    wid = cid * NS + sid
    ebase = wid * EW

    pltpu.sync_copy(src_hbm.at[pl.ds(ebase, EW)], srcf_v)
    pltpu.sync_copy(dst_hbm.at[pl.ds(ebase, EW)], dstf_v)

    zeros16 = jnp.zeros((L,), jnp.float32)

    for b in range(B):
        pltpu.sync_copy(alphas_hbm.at[pl.ds(b * N * AF, N * AF)], alpha_v)
        for h in range(N_HEADS):
            def zero_body(i, _):
                m_v[pl.ds(i * L, L)] = zeros16
                u_v[pl.ds(i * L, L)] = zeros16
                return 0
            lax.fori_loop(0, NPAD // L, zero_body, 0)

            def chunk_body(c, _):
                def vec_body(v, _):
                    off = c * CH1 + v * L
                    sidx = srcf_v[pl.ds(off, L)]
                    didx = dstf_v[pl.ds(off, L)]
                    a_s = plsc.load_gather(alpha_v, [sidx * AF + h])
                    a_d = plsc.load_gather(alpha_v, [didx * AF + (4 + h)])
                    s = a_s + a_d
                    s = jnp.where(s >= 0.0, s, s * 0.2)
                    p0 = jnp.exp(s)
                    p0b_v[pl.ds(v * L, L)] = p0
                    plsc.addupdate_scatter(u_v, [didx], p0)
                    cur = plsc.load_gather(m_v, [didx])
                    pending = s > cur

                    def wcond(carry):
                        return jnp.any(carry[0])

                    def wbody(carry):
                        pend, val, idx = carry
                        plsc.store_scatter(m_v, [idx], val, mask=pend)
                        chk = plsc.load_gather(m_v, [idx])
                        return (pend & (chk < val), val, idx)

                    lax.while_loop(wcond, wbody, (pending, s, didx))
                    return 0
                lax.fori_loop(0, CH1 // L, vec_body, 0)
                pltpu.sync_copy(
                    p0b_v,
                    p0_hbm.at[pl.ds((b * N_HEADS + h) * E + ebase + c * CH1,
                                    CH1)])
                return 0
            lax.fori_loop(0, NCH1, chunk_body, 0)

            slab_off = ((b * N_HEADS + h) * NW + wid) * NPAD
            pltpu.sync_copy(m_v, mslab_hbm.at[pl.ds(slab_off, NPAD)])
            pltpu.sync_copy(u_v, uslab_hbm.at[pl.ds(slab_off, NPAD)])


@functools.partial(
    pl.kernel,
    out_type=[
        jax.ShapeDtypeStruct((B * N_HEADS * E,), jnp.float32),
        jax.ShapeDtypeStruct((B * N_HEADS * NW * NPAD,), jnp.float32),
        jax.ShapeDtypeStruct((B * N_HEADS * NW * NPAD,), jnp.float32),
    ],
    mesh=_mesh,
    compiler_params=pltpu.CompilerParams(needs_layout_passes=False),
    scratch_types=[
        pltpu.VMEM((N * AF,), jnp.float32),
        pltpu.VMEM((EW,), jnp.int32),
        pltpu.VMEM((EW,), jnp.int32),
        pltpu.VMEM((NPAD,), jnp.float32),
        pltpu.VMEM((NPAD,), jnp.float32),
        pltpu.VMEM((CH1,), jnp.float32),
        pltpu.SemaphoreType.DMA,
    ],
)
def _sc_scores(alphas_hbm, src_hbm, dst_hbm, p0_hbm, mslab_hbm, uslab_hbm,
               alpha_v, srcf_v, dstf_v, m_v, u_v, p0b_v, sem):
    _sc_scores_body(alphas_hbm, src_hbm, dst_hbm, p0_hbm, mslab_hbm, uslab_hbm,
                    alpha_v, srcf_v, dstf_v, m_v, u_v, p0b_v, sem)


def _sc_combine_body(mslab_hbm, uslab_hbm, rden_hbm,
                     am_v, au_v, tb_v, rb_v, sem):
    del sem
    cid = lax.axis_index("c")
    sid = lax.axis_index("s")
    wid = cid * NS + sid
    nbase = wid * NB

    lanes = lax.iota(jnp.int32, L)

    for b in range(B):
        for h in range(N_HEADS):
            base = (b * N_HEADS + h) * NW * NPAD + nbase
            pltpu.sync_copy(mslab_hbm.at[pl.ds(base, NB)], am_v)
            pltpu.sync_copy(uslab_hbm.at[pl.ds(base, NB)], au_v)

            def t_body(t, _):
                off = (b * N_HEADS + h) * NW * NPAD + t * NPAD + nbase
                pltpu.sync_copy(mslab_hbm.at[pl.ds(off, NB)], tb_v)

                def vm_body(v, _):
                    sl = pl.ds(v * L, L)
                    am_v[sl] = jnp.maximum(am_v[sl], tb_v[sl])
                    return 0
                lax.fori_loop(0, NB // L, vm_body, 0)

                pltpu.sync_copy(uslab_hbm.at[pl.ds(off, NB)], tb_v)

                def vu_body(v, _):
                    sl = pl.ds(v * L, L)
                    au_v[sl] = au_v[sl] + tb_v[sl]
                    return 0
                lax.fori_loop(0, NB // L, vu_body, 0)
                return 0
            lax.fori_loop(1, NW, t_body, 0)

            def r_body(v, _):
                sl = pl.ds(v * L, L)
                denom = au_v[sl] + 1e-8 * jnp.exp(am_v[sl])
                r = 1.0 / denom
                idx = (lanes + v * L) * N_HEADS + h
                plsc.store_scatter(rb_v, [idx], r)
                return 0
            lax.fori_loop(0, NB // L, r_body, 0)
        pltpu.sync_copy(
            rb_v,
            rden_hbm.at[pl.ds(b * NPAD * N_HEADS + nbase * N_HEADS,
                              NB * N_HEADS)])


@functools.partial(
    pl.kernel,
    out_type=jax.ShapeDtypeStruct((B * NPAD * N_HEADS,), jnp.float32),
    mesh=_mesh,
    compiler_params=pltpu.CompilerParams(needs_layout_passes=False),
    scratch_types=[
        pltpu.VMEM((NB,), jnp.float32),
        pltpu.VMEM((NB,), jnp.float32),
        pltpu.VMEM((NB,), jnp.float32),
        pltpu.VMEM((NB * N_HEADS,), jnp.float32),
        pltpu.SemaphoreType.DMA,
    ],
)
def _sc_combine(mslab_hbm, uslab_hbm, rden_hbm, am_v, au_v, tb_v, rb_v, sem):
    _sc_combine_body(mslab_hbm, uslab_hbm, rden_hbm, am_v, au_v, tb_v, rb_v,
                     sem)


def _sc_wgt_body(p0_hbm, rden_hbm, dst_hbm, wq_hbm,
                 rden_v, dstf_v, pb_v, wb_v, sem):
    del sem
    cid = lax.axis_index("c")
    sid = lax.axis_index("s")
    wid = cid * NS + sid
    ebase = wid * EW

    pltpu.sync_copy(dst_hbm.at[pl.ds(ebase, EW)], dstf_v)

    for b in range(B):
        pltpu.sync_copy(
            rden_hbm.at[pl.ds(b * NPAD * N_HEADS, NPAD * N_HEADS)], rden_v)
        for h in range(N_HEADS):
            def chunk_body(c, _):
                off = (b * N_HEADS + h) * E + ebase + c * CH1
                pltpu.sync_copy(p0_hbm.at[pl.ds(off, CH1)], pb_v)

                def vec_body(v, _):
                    sl = pl.ds(v * L, L)
                    didx = dstf_v[pl.ds(c * CH1 + v * L, L)]
                    r = plsc.load_gather(rden_v, [didx * N_HEADS + h])
                    wb_v[sl] = pb_v[sl] * r
                    return 0
                lax.fori_loop(0, CH1 // L, vec_body, 0)
                pltpu.sync_copy(wb_v, wq_hbm.at[pl.ds(off, CH1)])
                return 0
            lax.fori_loop(0, NCH1, chunk_body, 0)


@functools.partial(
    pl.kernel,
    out_type=jax.ShapeDtypeStruct((B * N_HEADS * E,), jnp.float32),
    mesh=_mesh,
    compiler_params=pltpu.CompilerParams(needs_layout_passes=False),
    scratch_types=[
        pltpu.VMEM((NPAD * N_HEADS,), jnp.float32),
        pltpu.VMEM((EW,), jnp.int32),
        pltpu.VMEM((CH1,), jnp.float32),
        pltpu.VMEM((CH1,), jnp.float32),
        pltpu.SemaphoreType.DMA,
    ],
)
def _sc_wgt(p0_hbm, rden_hbm, dst_hbm, wq_hbm, rden_v, dstf_v, pb_v, wb_v,
            sem):
    _sc_wgt_body(p0_hbm, rden_hbm, dst_hbm, wq_hbm, rden_v, dstf_v, pb_v,
                 wb_v, sem)


def _sc_msg_body(h_hbm, wq_hbm, src4_hbm, dst4_hbm,
                 part_hbm,
                 src_v, dst_v, wqs_v, hrows_v, zb_v,
                 shared, sem):
    cid = lax.axis_index("c")
    sid = lax.axis_index("s")
    wid = cid * NS + sid

    z16 = jnp.zeros((L,), jnp.float32)

    def zz_body(i, _):
        r = i // (OUT_DIM // L)
        q = i % (OUT_DIM // L)
        zb_v[r, pl.ds(q * L, L)] = z16
        return 0
    lax.fori_loop(0, ZR * (OUT_DIM // L), zz_body, 0)

    for b in range(B):
        for z in range(RPT // ZR):
            pltpu.sync_copy(zb_v, shared.at[pl.ds(sid * RPT + z * ZR, ZR)])
        plsc.subcore_barrier()

        def sc_body(scc, _):
            pltpu.sync_copy(src4_hbm.at[wid, scc], src_v)
            pltpu.sync_copy(dst4_hbm.at[wid, scc], dst_v)
            for h in range(N_HEADS):
                pltpu.sync_copy(
                    wq_hbm.at[pl.ds((b * N_HEADS + h) * E + wid * EW
                                    + scc * SCH, SCH)],
                    wqs_v.at[pl.ds(h * SCH, SCH)])

            def ch_body(c2, _):
                pltpu.async_copy(
                    h_hbm.at[b].at[src_v.at[c2]], hrows_v, sem).wait()

                def e_body(e, _):
                    for h in range(N_HEADS):
                        wsp = plsc.load_gather(
                            wqs_v,
                            [jnp.full((L,), h * SCH, jnp.int32)
                             + c2 * CH + e])
                        for k in range(2):
                            vv = h * 2 + k
                            sl = pl.ds(vv * L, L)
                            hrows_v[e, sl] = hrows_v[e, sl] * wsp
                    return 0
                lax.fori_loop(0, CH, e_body, 0)

                pltpu.sync_copy(hrows_v, shared.at[dst_v.at[c2]], add=True)
                return 0
            lax.fori_loop(0, CPS, ch_body, 0)
            return 0
        lax.fori_loop(0, NSCH, sc_body, 0)

        plsc.subcore_barrier()
        part_row = (cid * B + b) * NPAD + sid * RPT
        pltpu.sync_copy(
            shared.at[pl.ds(sid * RPT, RPT)],
            part_hbm.at[pl.ds(part_row, RPT)])
        plsc.subcore_barrier()


@functools.partial(
    pl.kernel,
    out_type=jax.ShapeDtypeStruct((NC * B * NPAD, OUT_DIM), jnp.float32),
    mesh=_mesh,
    compiler_params=pltpu.CompilerParams(needs_layout_passes=False),
    scratch_types=[
        pltpu.VMEM((CPS, CH), jnp.int32),
        pltpu.VMEM((CPS, CH), jnp.int32),
        pltpu.VMEM((N_HEADS * SCH,), jnp.float32),
        pltpu.VMEM((CH, OUT_DIM), jnp.float32),
        pltpu.VMEM((ZR, OUT_DIM), jnp.float32),
        pltpu.VMEM_SHARED((NPAD, OUT_DIM), jnp.float32),
        pltpu.SemaphoreType.DMA,
    ],
)
def _sc_msg(h_hbm, wq_hbm, src4_hbm, dst4_hbm, part_hbm,
            src_v, dst_v, wqs_v, hrows_v, zb_v, shared, sem):
    _sc_msg_body(h_hbm, wq_hbm, src4_hbm, dst4_hbm, part_hbm,
                 src_v, dst_v, wqs_v, hrows_v, zb_v, shared, sem)


def _part_drain_shape(arr):
    return arr.reshape(NC, B, NPAD, OUT_DIM)


@jax.jit
def kernel(x, edge_index, W, attn):
    wt = W.T
    mask = jnp.repeat(jnp.eye(N_HEADS, dtype=jnp.float32), HEAD_DIM, axis=0)
    al_l = mask * attn[:, :HEAD_DIM].reshape(-1)[:, None]
    al_r = mask * attn[:, HEAD_DIM:].reshape(-1)[:, None]
    al = jnp.concatenate([al_l, al_r], axis=1)

    h, alphas = _tc_proj(x, wt, al)

    src = edge_index[0]
    dst = edge_index[1]

    p0, mslab, uslab = _sc_scores(alphas.reshape(B * N * AF), src, dst)
    rden = _sc_combine(mslab, uslab)
    wq = _sc_wgt(p0, rden, dst)
    parts_flat = _sc_msg(h, wq,
                         src.reshape(NW, NSCH, CPS, CH),
                         dst.reshape(NW, NSCH, CPS, CH))
    parts = _part_drain_shape(parts_flat)
    out = _tc_add(parts[0, :, :N], parts[1, :, :N])
    return out.reshape(B, N, OUT_DIM)


# speedup vs baseline: 60.9537x; 1.1043x over previous
import functools

import jax
import jax.numpy as jnp
from jax import lax
from jax.experimental import pallas as pl
from jax.experimental.pallas import tpu as pltpu
from jax.experimental.pallas import tpu_sc as plsc

IN_DIM = 128
OUT_DIM = 128
N_HEADS = 4
HEAD_DIM = OUT_DIM // N_HEADS
B = 2
N = 10000
E = 320000

NC = 2
NS = 16
NW = NC * NS
L = 16
EW = E // NW
NPAD = 10240
NPS = 12288
NB = NPS // NW

CH = 80
NCH = EW // CH
SCH = 2000
NSCH = EW // SCH
CPS = SCH // CH

CH1 = 400
NCH1 = EW // CH1

RPT = NPAD // NS
ZR = 64

AF = 2 * N_HEADS

_mesh = plsc.VectorSubcoreMesh(
    core_axis_name="c", subcore_axis_name="s", num_cores=NC, num_subcores=NS)


def _tc_proj_body(x_ref, wt_ref, al_ref, h_ref, a_ref):
    xb = x_ref[0]
    hb = jnp.dot(xb, wt_ref[...], preferred_element_type=jnp.float32)
    h_ref[0] = hb
    a_ref[0] = jnp.dot(hb, al_ref[...], preferred_element_type=jnp.float32)


def _tc_proj(x, wt, al):
    blk = 400
    grid = (B, N // blk)
    return pl.pallas_call(
        _tc_proj_body,
        grid=grid,
        in_specs=[
            pl.BlockSpec((1, blk, IN_DIM), lambda b, i: (b, i, 0)),
            pl.BlockSpec((IN_DIM, OUT_DIM), lambda b, i: (0, 0)),
            pl.BlockSpec((OUT_DIM, AF), lambda b, i: (0, 0)),
        ],
        out_specs=[
            pl.BlockSpec((1, blk, OUT_DIM), lambda b, i: (b, i, 0)),
            pl.BlockSpec((1, blk, AF), lambda b, i: (b, i, 0)),
        ],
        out_shape=[
            jax.ShapeDtypeStruct((B, N, OUT_DIM), jnp.float32),
            jax.ShapeDtypeStruct((B, N, AF), jnp.float32),
        ],
    )(x, wt, al)


def _tc_add_body(a_ref, b_ref, o_ref):
    o_ref[...] = a_ref[...] + b_ref[...]


def _tc_add(pa, pb):
    blk = 400
    grid = (B, N // blk)
    in_spec = pl.BlockSpec((1, blk, OUT_DIM), lambda b, i: (b, i, 0))
    return pl.pallas_call(
        _tc_add_body,
        grid=grid,
        in_specs=[in_spec, in_spec],
        out_specs=pl.BlockSpec((1, blk, OUT_DIM), lambda b, i: (b, i, 0)),
        out_shape=jax.ShapeDtypeStruct((B, N, OUT_DIM), jnp.float32),
    )(pa, pb)


def _sc_scores_body(alphas_hbm, src_hbm, dst_hbm,
                    p0_hbm, mslab_hbm, uslab_hbm,
                    alpha_v, srcf_v, dstf_v, m_v, u_v, p0b_v, sem):
    del sem
    cid = lax.axis_index("c")
    sid = lax.axis_index("s")
    wid = cid * NS + sid
    ebase = wid * EW

    pltpu.sync_copy(src_hbm.at[pl.ds(ebase, EW)], srcf_v)
    pltpu.sync_copy(dst_hbm.at[pl.ds(ebase, EW)], dstf_v)

    zeros16 = jnp.zeros((L,), jnp.float32)

    for b in range(B):
        pltpu.sync_copy(alphas_hbm.at[pl.ds(b * N * AF, N * AF)], alpha_v)
        for h in range(N_HEADS):
            def zero_body(i, _):
                m_v[pl.ds(i * L, L)] = zeros16
                u_v[pl.ds(i * L, L)] = zeros16
                return 0
            lax.fori_loop(0, NPS // L, zero_body, 0)

            def chunk_body(c, _):
                def vec_body(v, _):
                    off = c * CH1 + v * L
                    sidx = srcf_v[pl.ds(off, L)]
                    didx = dstf_v[pl.ds(off, L)]
                    a_s = plsc.load_gather(alpha_v, [sidx * AF + h])
                    a_d = plsc.load_gather(alpha_v, [didx * AF + (4 + h)])
                    s = a_s + a_d
                    s = jnp.where(s >= 0.0, s, s * 0.2)
                    p0 = jnp.exp(s)
                    p0b_v[pl.ds(v * L, L)] = p0
                    plsc.addupdate_scatter(u_v, [didx], p0)
                    cur = plsc.load_gather(m_v, [didx])
                    pending = s > cur

                    def wcond(carry):
                        return jnp.any(carry[0])

                    def wbody(carry):
                        pend, val, idx = carry
                        plsc.store_scatter(m_v, [idx], val, mask=pend)
                        chk = plsc.load_gather(m_v, [idx])
                        return (pend & (chk < val), val, idx)

                    lax.while_loop(wcond, wbody, (pending, s, didx))
                    return 0
                lax.fori_loop(0, CH1 // L, vec_body, 0)
                pltpu.sync_copy(
                    p0b_v,
                    p0_hbm.at[pl.ds((b * N_HEADS + h) * E + ebase + c * CH1,
                                    CH1)])
                return 0
            lax.fori_loop(0, NCH1, chunk_body, 0)

            slab_off = ((b * N_HEADS + h) * NW + wid) * NPS
            pltpu.sync_copy(m_v, mslab_hbm.at[pl.ds(slab_off, NPS)])
            pltpu.sync_copy(u_v, uslab_hbm.at[pl.ds(slab_off, NPS)])


@functools.partial(
    pl.kernel,
    out_type=[
        jax.ShapeDtypeStruct((B * N_HEADS * E,), jnp.float32),
        jax.ShapeDtypeStruct((B * N_HEADS * NW * NPS,), jnp.float32),
        jax.ShapeDtypeStruct((B * N_HEADS * NW * NPS,), jnp.float32),
    ],
    mesh=_mesh,
    compiler_params=pltpu.CompilerParams(needs_layout_passes=False),
    scratch_types=[
        pltpu.VMEM((N * AF,), jnp.float32),
        pltpu.VMEM((EW,), jnp.int32),
        pltpu.VMEM((EW,), jnp.int32),
        pltpu.VMEM((NPS,), jnp.float32),
        pltpu.VMEM((NPS,), jnp.float32),
        pltpu.VMEM((CH1,), jnp.float32),
        pltpu.SemaphoreType.DMA,
    ],
)
def _sc_scores(alphas_hbm, src_hbm, dst_hbm, p0_hbm, mslab_hbm, uslab_hbm,
               alpha_v, srcf_v, dstf_v, m_v, u_v, p0b_v, sem):
    _sc_scores_body(alphas_hbm, src_hbm, dst_hbm, p0_hbm, mslab_hbm, uslab_hbm,
                    alpha_v, srcf_v, dstf_v, m_v, u_v, p0b_v, sem)


def _sc_combine_body(mslab_hbm, uslab_hbm, rden_hbm,
                     am_v, au_v, tb_v, rb_v, sem):
    del sem
    cid = lax.axis_index("c")
    sid = lax.axis_index("s")
    wid = cid * NS + sid
    nbase = wid * NB

    lanes = lax.iota(jnp.int32, L)
    zeros16 = jnp.zeros((L,), jnp.float32)

    for b in range(B):
        for h in range(N_HEADS):
            r0 = (b * N_HEADS + h) * NW

            def zero_body(v, _):
                sl = pl.ds(v * L, L)
                am_v[sl] = zeros16
                au_v[sl] = zeros16
                return 0
            lax.fori_loop(0, NB // L, zero_body, 0)

            pltpu.sync_copy(
                mslab_hbm.at[pl.ds(r0, NW), pl.ds(nbase, NB)], tb_v)

            def tm_body(t, _):
                def vm_body(v, _):
                    sl = pl.ds(v * L, L)
                    am_v[sl] = jnp.maximum(am_v[sl], tb_v[t, sl])
                    return 0
                lax.fori_loop(0, NB // L, vm_body, 0)
                return 0
            lax.fori_loop(0, NW, tm_body, 0)

            pltpu.sync_copy(
                uslab_hbm.at[pl.ds(r0, NW), pl.ds(nbase, NB)], tb_v)

            def tu_body(t, _):
                def vu_body(v, _):
                    sl = pl.ds(v * L, L)
                    au_v[sl] = au_v[sl] + tb_v[t, sl]
                    return 0
                lax.fori_loop(0, NB // L, vu_body, 0)
                return 0
            lax.fori_loop(0, NW, tu_body, 0)

            def r_body(v, _):
                sl = pl.ds(v * L, L)
                denom = au_v[sl] + 1e-8 * jnp.exp(am_v[sl])
                r = 1.0 / denom
                idx = (lanes + v * L) * N_HEADS + h
                plsc.store_scatter(rb_v, [idx], r)
                return 0
            lax.fori_loop(0, NB // L, r_body, 0)
        pltpu.sync_copy(
            rb_v,
            rden_hbm.at[pl.ds(b * NPS * N_HEADS + nbase * N_HEADS,
                              NB * N_HEADS)])


@functools.partial(
    pl.kernel,
    out_type=jax.ShapeDtypeStruct((B * NPS * N_HEADS,), jnp.float32),
    mesh=_mesh,
    compiler_params=pltpu.CompilerParams(needs_layout_passes=False),
    scratch_types=[
        pltpu.VMEM((NB,), jnp.float32),
        pltpu.VMEM((NB,), jnp.float32),
        pltpu.VMEM((NW, NB), jnp.float32),
        pltpu.VMEM((NB * N_HEADS,), jnp.float32),
        pltpu.SemaphoreType.DMA,
    ],
)
def _sc_combine(mslab_hbm, uslab_hbm, rden_hbm, am_v, au_v, tb_v, rb_v, sem):
    _sc_combine_body(mslab_hbm, uslab_hbm, rden_hbm, am_v, au_v, tb_v, rb_v,
                     sem)


def _sc_wgt_body(p0_hbm, rden_hbm, dst_hbm, wq_hbm,
                 rden_v, dstf_v, pb_v, wb_v, sem):
    del sem
    cid = lax.axis_index("c")
    sid = lax.axis_index("s")
    wid = cid * NS + sid
    ebase = wid * EW

    pltpu.sync_copy(dst_hbm.at[pl.ds(ebase, EW)], dstf_v)

    for b in range(B):
        pltpu.sync_copy(
            rden_hbm.at[pl.ds(b * NPS * N_HEADS, NPS * N_HEADS)], rden_v)
        for h in range(N_HEADS):
            def chunk_body(c, _):
                off = (b * N_HEADS + h) * E + ebase + c * CH1
                pltpu.sync_copy(p0_hbm.at[pl.ds(off, CH1)], pb_v)

                def vec_body(v, _):
                    sl = pl.ds(v * L, L)
                    didx = dstf_v[pl.ds(c * CH1 + v * L, L)]
                    r = plsc.load_gather(rden_v, [didx * N_HEADS + h])
                    wb_v[sl] = pb_v[sl] * r
                    return 0
                lax.fori_loop(0, CH1 // L, vec_body, 0)
                pltpu.sync_copy(wb_v, wq_hbm.at[pl.ds(off, CH1)])
                return 0
            lax.fori_loop(0, NCH1, chunk_body, 0)


@functools.partial(
    pl.kernel,
    out_type=jax.ShapeDtypeStruct((B * N_HEADS * E,), jnp.float32),
    mesh=_mesh,
    compiler_params=pltpu.CompilerParams(needs_layout_passes=False),
    scratch_types=[
        pltpu.VMEM((NPS * N_HEADS,), jnp.float32),
        pltpu.VMEM((EW,), jnp.int32),
        pltpu.VMEM((CH1,), jnp.float32),
        pltpu.VMEM((CH1,), jnp.float32),
        pltpu.SemaphoreType.DMA,
    ],
)
def _sc_wgt(p0_hbm, rden_hbm, dst_hbm, wq_hbm, rden_v, dstf_v, pb_v, wb_v,
            sem):
    _sc_wgt_body(p0_hbm, rden_hbm, dst_hbm, wq_hbm, rden_v, dstf_v, pb_v,
                 wb_v, sem)


def _sc_msg_body(h_hbm, wq_hbm, src4_hbm, dst4_hbm,
                 part_hbm,
                 src_v, dst_v, wqs_v, hrows_v, zb_v,
                 shared, sem):
    cid = lax.axis_index("c")
    sid = lax.axis_index("s")
    wid = cid * NS + sid

    z16 = jnp.zeros((L,), jnp.float32)

    def zz_body(i, _):
        r = i // (OUT_DIM // L)
        q = i % (OUT_DIM // L)
        zb_v[r, pl.ds(q * L, L)] = z16
        return 0
    lax.fori_loop(0, ZR * (OUT_DIM // L), zz_body, 0)

    for b in range(B):
        for z in range(RPT // ZR):
            pltpu.sync_copy(zb_v, shared.at[pl.ds(sid * RPT + z * ZR, ZR)])
        plsc.subcore_barrier()

        def sc_body(scc, _):
            pltpu.sync_copy(src4_hbm.at[wid, scc], src_v)
            pltpu.sync_copy(dst4_hbm.at[wid, scc], dst_v)
            for h in range(N_HEADS):
                pltpu.sync_copy(
                    wq_hbm.at[pl.ds((b * N_HEADS + h) * E + wid * EW
                                    + scc * SCH, SCH)],
                    wqs_v.at[pl.ds(h * SCH, SCH)])

            def ch_body(c2, _):
                pltpu.async_copy(
                    h_hbm.at[b].at[src_v.at[c2]], hrows_v, sem).wait()

                def e_body(e, _):
                    for h in range(N_HEADS):
                        wsp = plsc.load_gather(
                            wqs_v,
                            [jnp.full((L,), h * SCH, jnp.int32)
                             + c2 * CH + e])
                        for k in range(2):
                            vv = h * 2 + k
                            sl = pl.ds(vv * L, L)
                            hrows_v[e, sl] = hrows_v[e, sl] * wsp
                    return 0
                lax.fori_loop(0, CH, e_body, 0)

                pltpu.sync_copy(hrows_v, shared.at[dst_v.at[c2]], add=True)
                return 0
            lax.fori_loop(0, CPS, ch_body, 0)
            return 0
        lax.fori_loop(0, NSCH, sc_body, 0)

        plsc.subcore_barrier()
        part_row = (cid * B + b) * NPAD + sid * RPT
        pltpu.sync_copy(
            shared.at[pl.ds(sid * RPT, RPT)],
            part_hbm.at[pl.ds(part_row, RPT)])
        plsc.subcore_barrier()


@functools.partial(
    pl.kernel,
    out_type=jax.ShapeDtypeStruct((NC * B * NPAD, OUT_DIM), jnp.float32),
    mesh=_mesh,
    compiler_params=pltpu.CompilerParams(needs_layout_passes=False),
    scratch_types=[
        pltpu.VMEM((CPS, CH), jnp.int32),
        pltpu.VMEM((CPS, CH), jnp.int32),
        pltpu.VMEM((N_HEADS * SCH,), jnp.float32),
        pltpu.VMEM((CH, OUT_DIM), jnp.float32),
        pltpu.VMEM((ZR, OUT_DIM), jnp.float32),
        pltpu.VMEM_SHARED((NPAD, OUT_DIM), jnp.float32),
        pltpu.SemaphoreType.DMA,
    ],
)
def _sc_msg(h_hbm, wq_hbm, src4_hbm, dst4_hbm, part_hbm,
            src_v, dst_v, wqs_v, hrows_v, zb_v, shared, sem):
    _sc_msg_body(h_hbm, wq_hbm, src4_hbm, dst4_hbm, part_hbm,
                 src_v, dst_v, wqs_v, hrows_v, zb_v, shared, sem)


def _part_drain_shape(arr):
    return arr.reshape(NC, B, NPAD, OUT_DIM)


@jax.jit
def kernel(x, edge_index, W, attn):
    wt = W.T
    mask = jnp.repeat(jnp.eye(N_HEADS, dtype=jnp.float32), HEAD_DIM, axis=0)
    al_l = mask * attn[:, :HEAD_DIM].reshape(-1)[:, None]
    al_r = mask * attn[:, HEAD_DIM:].reshape(-1)[:, None]
    al = jnp.concatenate([al_l, al_r], axis=1)

    h, alphas = _tc_proj(x, wt, al)

    src = edge_index[0]
    dst = edge_index[1]

    p0, mslab, uslab = _sc_scores(alphas.reshape(B * N * AF), src, dst)
    rden = _sc_combine(mslab.reshape(B * N_HEADS * NW, NPS),
                       uslab.reshape(B * N_HEADS * NW, NPS))
    wq = _sc_wgt(p0, rden, dst)
    parts_flat = _sc_msg(h, wq,
                         src.reshape(NW, NSCH, CPS, CH),
                         dst.reshape(NW, NSCH, CPS, CH))
    parts = _part_drain_shape(parts_flat)
    out = _tc_add(parts[0, :, :N], parts[1, :, :N])
    return out.reshape(B, N, OUT_DIM)
